# baseline (speedup 1.0000x reference)
import jax
import jax.numpy as jnp
from jax.experimental import pallas as pl

_BN = 1024


def _tmk_block(x_ref, e_ref, pts_t_ref, c_ref, out_ref):
    D, M = pts_t_ref.shape
    b = jnp.dot(
        x_ref[...],
        e_ref[...],
        preferred_element_type=jnp.float32,
        precision=jax.lax.Precision.HIGHEST,
    )
    acc = None
    for d in range(D):
        t = jnp.abs(b[:, d * M : (d + 1) * M] - pts_t_ref[d : d + 1, :])
        acc = t if acc is None else acc + t
    out_ref[...] = jnp.dot(
        jnp.exp(-acc), c_ref[...], preferred_element_type=jnp.float32
    )


def kernel(input, pts_set, chol_inv):
    N, D = input.shape
    M = pts_set.shape[0]
    pts_t = pts_set.T
    e = jnp.repeat(jnp.eye(D, dtype=jnp.float32), M, axis=1)
    return pl.pallas_call(
        _tmk_block,
        grid=(N // _BN,),
        in_specs=[
            pl.BlockSpec((_BN, D), lambda i: (i, 0)),
            pl.BlockSpec((D, D * M), lambda i: (0, 0)),
            pl.BlockSpec((D, M), lambda i: (0, 0)),
            pl.BlockSpec((M, M), lambda i: (0, 0)),
        ],
        out_specs=pl.BlockSpec((_BN, M), lambda i: (i, 0)),
        out_shape=jax.ShapeDtypeStruct((N, M), jnp.float32),
    )(input, e, pts_t, chol_inv)


# speedup vs baseline: 5.2278x; 5.2278x over previous
import jax
import jax.numpy as jnp
from jax.experimental import pallas as pl

_BN = 1024
_C = 128


def _tmk_block(xt_ref, ptsb_ref, c_ref, out_ref):
    D = xt_ref.shape[0]
    M = c_ref.shape[0]
    c = c_ref[...]
    for j in range(_BN // _C):
        acc = None
        for d in range(D):
            xr = xt_ref[d : d + 1, j * _C : (j + 1) * _C]
            pb = ptsb_ref[d * M : (d + 1) * M, :]
            t = jnp.abs(pb - xr)
            acc = t if acc is None else acc + t
        kt = jnp.exp(-acc)
        out_ref[j * _C : (j + 1) * _C, :] = jax.lax.dot_general(
            kt, c, (((0,), (0,)), ((), ())), preferred_element_type=jnp.float32
        )


def kernel(input, pts_set, chol_inv):
    N, D = input.shape
    M = pts_set.shape[0]
    xt = input.T
    pts_b = jnp.broadcast_to(pts_set.T[:, :, None], (D, M, _C)).reshape(D * M, _C)
    return pl.pallas_call(
        _tmk_block,
        grid=(N // _BN,),
        in_specs=[
            pl.BlockSpec((D, _BN), lambda i: (0, i)),
            pl.BlockSpec((D * M, _C), lambda i: (0, 0)),
            pl.BlockSpec((M, M), lambda i: (0, 0)),
        ],
        out_specs=pl.BlockSpec((_BN, M), lambda i: (i, 0)),
        out_shape=jax.ShapeDtypeStruct((N, M), jnp.float32),
    )(xt, pts_b, chol_inv)


# speedup vs baseline: 5.2354x; 1.0015x over previous
import jax
import jax.numpy as jnp
from jax.experimental import pallas as pl
from jax.experimental.pallas import tpu as pltpu

_BN = 1024
_C = 128


def _tmk_block(xt_ref, ptsb_ref, c_ref, out_ref):
    D = xt_ref.shape[0]
    M = c_ref.shape[0]
    c = c_ref[...]
    for j in range(_BN // _C):
        acc = None
        for d in range(D):
            xr = xt_ref[d : d + 1, j * _C : (j + 1) * _C]
            pb = ptsb_ref[d * M : (d + 1) * M, :]
            t = jnp.abs(pb - xr)
            acc = t if acc is None else acc + t
        kt = jnp.exp(-acc)
        out_ref[j * _C : (j + 1) * _C, :] = jax.lax.dot_general(
            kt, c, (((0,), (0,)), ((), ())), preferred_element_type=jnp.float32
        )


def kernel(input, pts_set, chol_inv):
    N, D = input.shape
    M = pts_set.shape[0]
    xt = input.T
    pts_b = jnp.broadcast_to(pts_set.T[:, :, None], (D, M, _C)).reshape(D * M, _C)
    return pl.pallas_call(
        _tmk_block,
        grid=(N // _BN,),
        in_specs=[
            pl.BlockSpec((D, _BN), lambda i: (0, i)),
            pl.BlockSpec((D * M, _C), lambda i: (0, 0)),
            pl.BlockSpec((M, M), lambda i: (0, 0)),
        ],
        out_specs=pl.BlockSpec((_BN, M), lambda i: (i, 0)),
        out_shape=jax.ShapeDtypeStruct((N, M), jnp.float32),
        compiler_params=pltpu.CompilerParams(
            dimension_semantics=("parallel",),
        ),
    )(xt, pts_b, chol_inv)
